# baseline (speedup 1.0000x reference)
import functools

import jax
import jax.numpy as jnp
from jax import lax
from jax.experimental import pallas as pl
from jax.experimental.pallas import tpu as pltpu
from jax.experimental.pallas import tpu_sc as plsc

N = 10000
NPAD = 10112
ROWS_PER_SUB = NPAD // 16
E = 160000
EROWS = E // 128
NW = 32
ROWS_C0 = 52
ROWS_C1 = 26
TAIL = 8
W = 16
BLK = 13
ROWS_MAX = max(ROWS_C0, ROWS_C1)


def _sc_agg(u, ei3, zeros):
    mesh = plsc.VectorSubcoreMesh(core_axis_name="c", subcore_axis_name="s")

    @functools.partial(
        pl.kernel,
        out_type=jax.ShapeDtypeStruct((2, NPAD, W), jnp.float32),
        mesh=mesh,
        compiler_params=pltpu.CompilerParams(use_tc_tiling_on_sc=False),
        scratch_types=[
            pltpu.VMEM((ROWS_MAX, 128), jnp.int32),
            pltpu.VMEM((ROWS_MAX, 128), jnp.int32),
            pltpu.VMEM((TAIL,), jnp.int32),
            pltpu.VMEM((TAIL,), jnp.int32),
            pltpu.VMEM((ROWS_MAX * 128, W), jnp.float32),
            pltpu.VMEM((TAIL, W), jnp.float32),
            pltpu.VMEM_SHARED((NPAD, W), jnp.float32),
            pltpu.SemaphoreType.DMA,
            pltpu.SemaphoreType.DMA((2,)),
            pltpu.SemaphoreType.DMA((2,)),
        ],
    )
    def agg(u_hbm, ei_hbm, z_hbm, parts_hbm, idx_s, idx_d, tidx_s, tidx_d,
            rows, trows, acc, sem_i, sem_g, sem_s):
        cid = lax.axis_index("c")
        sid = lax.axis_index("s")
        wid = sid * 2 + cid
        stripe = pl.ds(sid * ROWS_PER_SUB, ROWS_PER_SUB)
        erow0 = jnp.where(cid == 0, sid * ROWS_C0,
                          16 * ROWS_C0 + sid * ROWS_C1)
        nblk = jnp.where(cid == 0, ROWS_C0 // BLK, ROWS_C1 // BLK)
        trow = EROWS - 2 + wid // 16
        tcol = lax.rem(wid, 16) * TAIL

        h0 = pltpu.async_copy(z_hbm.at[stripe], acc.at[stripe], sem_i)
        h1 = pltpu.async_copy(ei_hbm.at[0, pl.ds(erow0, ROWS_MAX)], idx_s,
                              sem_i)
        h2 = pltpu.async_copy(ei_hbm.at[1, pl.ds(erow0, ROWS_MAX)], idx_d,
                              sem_i)
        h3 = pltpu.async_copy(ei_hbm.at[0, trow, pl.ds(tcol, TAIL)], tidx_s,
                              sem_i)
        h4 = pltpu.async_copy(ei_hbm.at[1, trow, pl.ds(tcol, TAIL)], tidx_d,
                              sem_i)
        h0.wait(); h1.wait(); h2.wait(); h3.wait(); h4.wait()
        plsc.subcore_barrier()

        def fire_gathers(g, slot):
            @pl.loop(0, BLK)
            def _(i):
                j = g * BLK + i
                pltpu.async_copy(u_hbm.at[idx_s.at[j]],
                                 rows.at[pl.ds(j * 128, 128)],
                                 sem_g.at[slot])

        def drain_gathers(slot):
            @pl.loop(0, BLK)
            def _(i):
                pltpu.make_async_copy(u_hbm.at[idx_s.at[0]],
                                      rows.at[pl.ds(0, 128)],
                                      sem_g.at[slot]).wait()

        def fire_scatters(g, slot):
            @pl.loop(0, BLK)
            def _(i):
                j = g * BLK + i
                pltpu.async_copy(rows.at[pl.ds(j * 128, 128)],
                                 acc.at[idx_d.at[j]], sem_s.at[slot],
                                 add=True)

        def drain_scatters(slot):
            @pl.loop(0, BLK)
            def _(i):
                pltpu.make_async_copy(rows.at[pl.ds(0, 128)],
                                      acc.at[idx_d.at[0]],
                                      sem_s.at[slot]).wait()

        fire_gathers(0, 0)

        @pl.loop(0, nblk)
        def _(g):
            slot = lax.rem(g, 2)
            nslot = lax.rem(g + 1, 2)

            @pl.when(g + 1 < nblk)
            def _():
                fire_gathers(g + 1, nslot)

            drain_gathers(slot)

            @pl.when(g >= 2)
            def _():
                drain_scatters(slot)

            fire_scatters(g, slot)

        pltpu.sync_copy(u_hbm.at[tidx_s], trows)
        pltpu.sync_copy(trows, acc.at[tidx_d], add=True)

        drain_scatters(0)
        drain_scatters(1)

        plsc.subcore_barrier()
        pltpu.sync_copy(acc.at[stripe], parts_hbm.at[cid, stripe])

    return agg(u, ei3, zeros)


def _tc_call(body, out_shapes, *args):
    return pl.pallas_call(body, out_shape=out_shapes)(*args)


def _mm_body(x_ref, w_ref, o_ref):
    o_ref[0:N, :] = jnp.dot(x_ref[...], w_ref[...],
                            preferred_element_type=jnp.float32)
    o_ref[N:NPAD, :] = jnp.zeros((NPAD - N, W), jnp.float32)


def _deg_body(dp_ref, z_ref, dinv_ref, u_ref):
    deg = dp_ref[0] + dp_ref[1] + 1.0
    dinv = lax.rsqrt(deg)
    dinv_ref[...] = dinv
    u_ref[...] = dinv * z_ref[...]


def _layer_body(p_ref, u_ref, dinv_ref, w_ref, b_ref, un_ref):
    h = jnp.tanh(dinv_ref[...] * (p_ref[0] + p_ref[1] + u_ref[...])
                 + b_ref[...])
    un_ref[...] = dinv_ref[...] * jnp.dot(h, w_ref[...],
                                          preferred_element_type=jnp.float32)


def _final_body(p_ref, u_ref, dinv_ref, b_ref, wc_ref, bc_ref, out_ref, h_ref):
    h = jnp.tanh(dinv_ref[...] * (p_ref[0] + p_ref[1] + u_ref[...])
                 + b_ref[...])
    h_ref[...] = h[0:N, 0:2]
    out_ref[...] = jnp.dot(h[0:N], wc_ref[...],
                           preferred_element_type=jnp.float32) + bc_ref[...]


def _padw(w):
    return jnp.pad(w, ((0, 16 - w.shape[0]), (0, 16 - w.shape[1])))


def kernel(x, edge_index, W1, b1, W2, b2, W3, b3, Wc, bc):
    f32 = jnp.float32
    ei3 = edge_index.reshape(2, EROWS, 128)

    W1p = jnp.pad(W1, ((0, 0), (0, W - W1.shape[1])))
    W2p = _padw(W2)
    W3p = _padw(W3)
    Wcp = jnp.pad(Wc, ((0, 16 - Wc.shape[0]), (0, 0)))
    b1p = jnp.pad(b1, (0, W - b1.shape[0])).reshape(1, W)
    b2p = jnp.pad(b2, (0, W - b2.shape[0])).reshape(1, W)
    b3p = jnp.pad(b3, (0, W - b3.shape[0])).reshape(1, W)
    bcp = bc.reshape(1, bc.shape[0])

    zeros = jnp.zeros((NPAD, W), f32)
    ones = jnp.ones((NPAD, W), f32)

    sds = jax.ShapeDtypeStruct

    z1 = _tc_call(_mm_body, sds((NPAD, W), f32), x, W1p)

    degp = _sc_agg(ones, ei3, zeros)
    dinv, u1 = _tc_call(_deg_body, [sds((NPAD, W), f32), sds((NPAD, W), f32)],
                        degp, z1)

    p1 = _sc_agg(u1, ei3, zeros)
    u2 = _tc_call(_layer_body, sds((NPAD, W), f32), p1, u1, dinv, W2p, b1p)

    p2 = _sc_agg(u2, ei3, zeros)
    u3 = _tc_call(_layer_body, sds((NPAD, W), f32), p2, u2, dinv, W3p, b2p)

    p3 = _sc_agg(u3, ei3, zeros)
    out, h = _tc_call(
        _final_body,
        [sds((N, bc.shape[0]), f32), sds((N, 2), f32)],
        p3, u3, dinv, b3p, Wcp, bcp)

    return (out, h)


# speedup vs baseline: 1.0583x; 1.0583x over previous
import functools

import jax
import jax.numpy as jnp
from jax import lax
from jax.experimental import pallas as pl
from jax.experimental.pallas import tpu as pltpu
from jax.experimental.pallas import tpu_sc as plsc

N = 10000
NPAD = 10112
ROWS_PER_SUB = NPAD // 16
E = 160000
EROWS = E // 128
NW = 32
EROWS_PER_W = 39
TAIL = 8
W = 16
BLK = 13
NBLK = EROWS_PER_W // BLK

_MESH = plsc.VectorSubcoreMesh(core_axis_name="c", subcore_axis_name="s")
_CP = pltpu.CompilerParams(use_tc_tiling_on_sc=False)


def _fill_rows(buf, nrows, row_val):
    @pl.loop(0, nrows)
    def _(i):
        buf[i, :] = row_val


def _worker_ids():
    cid = lax.axis_index("c")
    sid = lax.axis_index("s")
    wid = sid * 2 + cid
    return cid, sid, wid


def _sc_agg(u, ei3):
    @functools.partial(
        pl.kernel,
        out_type=jax.ShapeDtypeStruct((2, NPAD, W), jnp.float32),
        mesh=_MESH,
        compiler_params=_CP,
        scratch_types=[
            pltpu.VMEM((EROWS_PER_W, 128), jnp.int32),
            pltpu.VMEM((EROWS_PER_W, 128), jnp.int32),
            pltpu.VMEM((TAIL,), jnp.int32),
            pltpu.VMEM((TAIL,), jnp.int32),
            pltpu.VMEM((EROWS_PER_W * 128, W), jnp.float32),
            pltpu.VMEM((TAIL, W), jnp.float32),
            pltpu.VMEM((ROWS_PER_SUB, W), jnp.float32),
            pltpu.VMEM_SHARED((NPAD, W), jnp.float32),
            pltpu.SemaphoreType.DMA,
            pltpu.SemaphoreType.DMA((2,)),
            pltpu.SemaphoreType.DMA((2,)),
        ],
    )
    def agg(u_hbm, ei_hbm, parts_hbm, idx_s, idx_d, tidx_s, tidx_d,
            rows, trows, zbuf, acc, sem_i, sem_g, sem_s):
        cid, sid, wid = _worker_ids()
        stripe = pl.ds(sid * ROWS_PER_SUB, ROWS_PER_SUB)
        erow0 = wid * EROWS_PER_W
        trow = EROWS - 2 + wid // 16
        tcol = lax.rem(wid, 16) * TAIL

        h1 = pltpu.async_copy(ei_hbm.at[0, pl.ds(erow0, EROWS_PER_W)], idx_s,
                              sem_i)
        h2 = pltpu.async_copy(ei_hbm.at[1, pl.ds(erow0, EROWS_PER_W)], idx_d,
                              sem_i)
        h3 = pltpu.async_copy(ei_hbm.at[0, trow, pl.ds(tcol, TAIL)], tidx_s,
                              sem_i)
        h4 = pltpu.async_copy(ei_hbm.at[1, trow, pl.ds(tcol, TAIL)], tidx_d,
                              sem_i)
        _fill_rows(zbuf, ROWS_PER_SUB, jnp.zeros((W,), jnp.float32))
        pltpu.sync_copy(zbuf, acc.at[stripe])
        h1.wait(); h2.wait(); h3.wait(); h4.wait()
        plsc.subcore_barrier()

        def fire_gathers(g, slot):
            @pl.loop(0, BLK)
            def _(i):
                j = g * BLK + i
                pltpu.async_copy(u_hbm.at[idx_s.at[j]],
                                 rows.at[pl.ds(j * 128, 128)],
                                 sem_g.at[slot])

        def drain_gathers(slot):
            @pl.loop(0, BLK)
            def _(i):
                pltpu.make_async_copy(u_hbm.at[idx_s.at[0]],
                                      rows.at[pl.ds(0, 128)],
                                      sem_g.at[slot]).wait()

        def fire_scatters(g, slot):
            @pl.loop(0, BLK)
            def _(i):
                j = g * BLK + i
                pltpu.async_copy(rows.at[pl.ds(j * 128, 128)],
                                 acc.at[idx_d.at[j]], sem_s.at[slot],
                                 add=True)

        def drain_scatters(slot):
            @pl.loop(0, BLK)
            def _(i):
                pltpu.make_async_copy(rows.at[pl.ds(0, 128)],
                                      acc.at[idx_d.at[0]],
                                      sem_s.at[slot]).wait()

        fire_gathers(0, 0)

        @pl.loop(0, NBLK)
        def _(g):
            slot = lax.rem(g, 2)
            nslot = lax.rem(g + 1, 2)

            @pl.when(g + 1 < NBLK)
            def _():
                fire_gathers(g + 1, nslot)

            drain_gathers(slot)

            @pl.when(g >= 2)
            def _():
                drain_scatters(slot)

            fire_scatters(g, slot)

        pltpu.sync_copy(u_hbm.at[tidx_s], trows)
        pltpu.sync_copy(trows, acc.at[tidx_d], add=True)

        drain_scatters(0)
        drain_scatters(1)

        plsc.subcore_barrier()
        pltpu.sync_copy(acc.at[stripe], parts_hbm.at[cid, stripe])

    return agg(u, ei3)


def _sc_deg(ei3):
    @functools.partial(
        pl.kernel,
        out_type=jax.ShapeDtypeStruct((2, NPAD, W), jnp.float32),
        mesh=_MESH,
        compiler_params=_CP,
        scratch_types=[
            pltpu.VMEM((EROWS_PER_W, 128), jnp.int32),
            pltpu.VMEM((TAIL,), jnp.int32),
            pltpu.VMEM((128, W), jnp.float32),
            pltpu.VMEM((ROWS_PER_SUB, W), jnp.float32),
            pltpu.VMEM_SHARED((NPAD, W), jnp.float32),
            pltpu.SemaphoreType.DMA,
            pltpu.SemaphoreType.DMA,
        ],
    )
    def deg(ei_hbm, parts_hbm, idx_d, tidx_d, ones, zbuf, acc, sem_i, sem_s):
        cid, sid, wid = _worker_ids()
        stripe = pl.ds(sid * ROWS_PER_SUB, ROWS_PER_SUB)
        erow0 = wid * EROWS_PER_W
        trow = EROWS - 2 + wid // 16
        tcol = lax.rem(wid, 16) * TAIL

        h2 = pltpu.async_copy(ei_hbm.at[1, pl.ds(erow0, EROWS_PER_W)], idx_d,
                              sem_i)
        h4 = pltpu.async_copy(ei_hbm.at[1, trow, pl.ds(tcol, TAIL)], tidx_d,
                              sem_i)
        _fill_rows(ones, 128, jnp.ones((W,), jnp.float32))
        _fill_rows(zbuf, ROWS_PER_SUB, jnp.zeros((W,), jnp.float32))
        pltpu.sync_copy(zbuf, acc.at[stripe])
        h2.wait(); h4.wait()
        plsc.subcore_barrier()

        @pl.loop(0, EROWS_PER_W)
        def _(j):
            pltpu.async_copy(ones, acc.at[idx_d.at[j]], sem_s, add=True)

        pltpu.sync_copy(ones.at[pl.ds(0, TAIL)], acc.at[tidx_d], add=True)

        @pl.loop(0, EROWS_PER_W)
        def _(j):
            pltpu.make_async_copy(ones, acc.at[idx_d.at[0]], sem_s).wait()

        plsc.subcore_barrier()
        pltpu.sync_copy(acc.at[stripe], parts_hbm.at[cid, stripe])

    return deg(ei3)


def _tc_call(body, out_shapes, *args):
    return pl.pallas_call(body, out_shape=out_shapes)(*args)


def _mm_body(x_ref, w_ref, o_ref):
    o_ref[0:N, :] = jnp.dot(x_ref[...], w_ref[...],
                            preferred_element_type=jnp.float32)
    o_ref[N:NPAD, :] = jnp.zeros((NPAD - N, W), jnp.float32)


def _deg_body(dp_ref, z_ref, dinv_ref, u_ref):
    deg = dp_ref[0] + dp_ref[1] + 1.0
    dinv = lax.rsqrt(deg)
    dinv_ref[...] = dinv
    u_ref[...] = dinv * z_ref[...]


def _layer_body(p_ref, u_ref, dinv_ref, w_ref, b_ref, un_ref):
    h = jnp.tanh(dinv_ref[...] * (p_ref[0] + p_ref[1] + u_ref[...])
                 + b_ref[...])
    un_ref[...] = dinv_ref[...] * jnp.dot(h, w_ref[...],
                                          preferred_element_type=jnp.float32)


def _final_body(p_ref, u_ref, dinv_ref, b_ref, wc_ref, bc_ref, out_ref, h_ref):
    h = jnp.tanh(dinv_ref[...] * (p_ref[0] + p_ref[1] + u_ref[...])
                 + b_ref[...])
    h_ref[...] = h[0:N, 0:2]
    out_ref[...] = jnp.dot(h[0:N], wc_ref[...],
                           preferred_element_type=jnp.float32) + bc_ref[...]


def _padw(w):
    return jnp.pad(w, ((0, 16 - w.shape[0]), (0, 16 - w.shape[1])))


def kernel(x, edge_index, W1, b1, W2, b2, W3, b3, Wc, bc):
    f32 = jnp.float32
    ei3 = edge_index.reshape(2, EROWS, 128)

    W1p = jnp.pad(W1, ((0, 0), (0, W - W1.shape[1])))
    W2p = _padw(W2)
    W3p = _padw(W3)
    Wcp = jnp.pad(Wc, ((0, 16 - Wc.shape[0]), (0, 0)))
    b1p = jnp.pad(b1, (0, W - b1.shape[0])).reshape(1, W)
    b2p = jnp.pad(b2, (0, W - b2.shape[0])).reshape(1, W)
    b3p = jnp.pad(b3, (0, W - b3.shape[0])).reshape(1, W)
    bcp = bc.reshape(1, bc.shape[0])

    sds = jax.ShapeDtypeStruct

    z1 = _tc_call(_mm_body, sds((NPAD, W), f32), x, W1p)

    degp = _sc_deg(ei3)
    dinv, u1 = _tc_call(_deg_body, [sds((NPAD, W), f32), sds((NPAD, W), f32)],
                        degp, z1)

    p1 = _sc_agg(u1, ei3)
    u2 = _tc_call(_layer_body, sds((NPAD, W), f32), p1, u1, dinv, W2p, b1p)

    p2 = _sc_agg(u2, ei3)
    u3 = _tc_call(_layer_body, sds((NPAD, W), f32), p2, u2, dinv, W3p, b2p)

    p3 = _sc_agg(u3, ei3)
    out, h = _tc_call(
        _final_body,
        [sds((N, bc.shape[0]), f32), sds((N, 2), f32)],
        p3, u3, dinv, b3p, Wcp, bcp)

    return (out, h)


# speedup vs baseline: 1.1922x; 1.1265x over previous
import functools

import jax
import jax.numpy as jnp
from jax import lax
from jax.experimental import pallas as pl
from jax.experimental.pallas import tpu as pltpu
from jax.experimental.pallas import tpu_sc as plsc

N = 10000
NPAD = 10240
ROWS_PER_SUB = NPAD // 16
ROWS_PER_WKR = NPAD // 32
E = 160000
NUM_OUT = 8
EROWS = E // 128
NW = 32
EROWS_PER_W = 39
TAIL = 8
W = 16
BLK = 13
NBLK = EROWS_PER_W // BLK

_MESH = plsc.VectorSubcoreMesh(core_axis_name="c", subcore_axis_name="s")
_CP = pltpu.CompilerParams(use_tc_tiling_on_sc=False)


def _fill_rows(buf, nrows, row_val):
    @pl.loop(0, nrows)
    def _(i):
        buf[i, :] = row_val


def _worker_ids():
    cid = lax.axis_index("c")
    sid = lax.axis_index("s")
    wid = sid * 2 + cid
    return cid, sid, wid


def _sc_agg(u, ei3):
    @functools.partial(
        pl.kernel,
        out_type=jax.ShapeDtypeStruct((2, NPAD, W), jnp.float32),
        mesh=_MESH,
        compiler_params=_CP,
        scratch_types=[
            pltpu.VMEM((EROWS_PER_W, 128), jnp.int32),
            pltpu.VMEM((EROWS_PER_W, 128), jnp.int32),
            pltpu.VMEM((TAIL,), jnp.int32),
            pltpu.VMEM((TAIL,), jnp.int32),
            pltpu.VMEM((EROWS_PER_W * 128, W), jnp.float32),
            pltpu.VMEM((TAIL, W), jnp.float32),
            pltpu.VMEM((ROWS_PER_SUB, W), jnp.float32),
            pltpu.VMEM_SHARED((NPAD, W), jnp.float32),
            pltpu.SemaphoreType.DMA,
            pltpu.SemaphoreType.DMA((2,)),
            pltpu.SemaphoreType.DMA((2,)),
        ],
    )
    def agg(u_hbm, ei_hbm, parts_hbm, idx_s, idx_d, tidx_s, tidx_d,
            rows, trows, zbuf, acc, sem_i, sem_g, sem_s):
        cid, sid, wid = _worker_ids()
        stripe = pl.ds(sid * ROWS_PER_SUB, ROWS_PER_SUB)
        erow0 = wid * EROWS_PER_W
        trow = EROWS - 2 + wid // 16
        tcol = lax.rem(wid, 16) * TAIL

        h1 = pltpu.async_copy(ei_hbm.at[0, pl.ds(erow0, EROWS_PER_W)], idx_s,
                              sem_i)
        h2 = pltpu.async_copy(ei_hbm.at[1, pl.ds(erow0, EROWS_PER_W)], idx_d,
                              sem_i)
        h3 = pltpu.async_copy(ei_hbm.at[0, trow, pl.ds(tcol, TAIL)], tidx_s,
                              sem_i)
        h4 = pltpu.async_copy(ei_hbm.at[1, trow, pl.ds(tcol, TAIL)], tidx_d,
                              sem_i)
        _fill_rows(zbuf, ROWS_PER_SUB, jnp.zeros((W,), jnp.float32))
        pltpu.sync_copy(zbuf, acc.at[stripe])
        h1.wait(); h2.wait(); h3.wait(); h4.wait()
        plsc.subcore_barrier()

        def fire_gathers(g, slot):
            @pl.loop(0, BLK)
            def _(i):
                j = g * BLK + i
                pltpu.async_copy(u_hbm.at[idx_s.at[j]],
                                 rows.at[pl.ds(j * 128, 128)],
                                 sem_g.at[slot])

        def drain_gathers(slot):
            @pl.loop(0, BLK)
            def _(i):
                pltpu.make_async_copy(u_hbm.at[idx_s.at[0]],
                                      rows.at[pl.ds(0, 128)],
                                      sem_g.at[slot]).wait()

        def fire_scatters(g, slot):
            @pl.loop(0, BLK)
            def _(i):
                j = g * BLK + i
                pltpu.async_copy(rows.at[pl.ds(j * 128, 128)],
                                 acc.at[idx_d.at[j]], sem_s.at[slot],
                                 add=True)

        def drain_scatters(slot):
            @pl.loop(0, BLK)
            def _(i):
                pltpu.make_async_copy(rows.at[pl.ds(0, 128)],
                                      acc.at[idx_d.at[0]],
                                      sem_s.at[slot]).wait()

        fire_gathers(0, 0)

        @pl.loop(0, NBLK)
        def _(g):
            slot = lax.rem(g, 2)
            nslot = lax.rem(g + 1, 2)

            @pl.when(g + 1 < NBLK)
            def _():
                fire_gathers(g + 1, nslot)

            drain_gathers(slot)

            @pl.when(g >= 2)
            def _():
                drain_scatters(slot)

            fire_scatters(g, slot)

        pltpu.sync_copy(u_hbm.at[tidx_s], trows)
        pltpu.sync_copy(trows, acc.at[tidx_d], add=True)

        drain_scatters(0)
        drain_scatters(1)

        plsc.subcore_barrier()
        pltpu.sync_copy(acc.at[stripe], parts_hbm.at[cid, stripe])

    return agg(u, ei3)


def _sc_deg(ei3):
    @functools.partial(
        pl.kernel,
        out_type=jax.ShapeDtypeStruct((2, NPAD, W), jnp.float32),
        mesh=_MESH,
        compiler_params=_CP,
        scratch_types=[
            pltpu.VMEM((EROWS_PER_W, 128), jnp.int32),
            pltpu.VMEM((TAIL,), jnp.int32),
            pltpu.VMEM((128, W), jnp.float32),
            pltpu.VMEM((ROWS_PER_SUB, W), jnp.float32),
            pltpu.VMEM_SHARED((NPAD, W), jnp.float32),
            pltpu.SemaphoreType.DMA,
            pltpu.SemaphoreType.DMA,
        ],
    )
    def deg(ei_hbm, parts_hbm, idx_d, tidx_d, ones, zbuf, acc, sem_i, sem_s):
        cid, sid, wid = _worker_ids()
        stripe = pl.ds(sid * ROWS_PER_SUB, ROWS_PER_SUB)
        erow0 = wid * EROWS_PER_W
        trow = EROWS - 2 + wid // 16
        tcol = lax.rem(wid, 16) * TAIL

        h2 = pltpu.async_copy(ei_hbm.at[1, pl.ds(erow0, EROWS_PER_W)], idx_d,
                              sem_i)
        h4 = pltpu.async_copy(ei_hbm.at[1, trow, pl.ds(tcol, TAIL)], tidx_d,
                              sem_i)
        _fill_rows(ones, 128, jnp.ones((W,), jnp.float32))
        _fill_rows(zbuf, ROWS_PER_SUB, jnp.zeros((W,), jnp.float32))
        pltpu.sync_copy(zbuf, acc.at[stripe])
        h2.wait(); h4.wait()
        plsc.subcore_barrier()

        @pl.loop(0, EROWS_PER_W)
        def _(j):
            pltpu.async_copy(ones, acc.at[idx_d.at[j]], sem_s, add=True)

        pltpu.sync_copy(ones.at[pl.ds(0, TAIL)], acc.at[tidx_d], add=True)

        @pl.loop(0, EROWS_PER_W)
        def _(j):
            pltpu.make_async_copy(ones, acc.at[idx_d.at[0]], sem_s).wait()

        plsc.subcore_barrier()
        pltpu.sync_copy(acc.at[stripe], parts_hbm.at[cid, stripe])

    return deg(ei3)


def _nrsqrt(s):
    i = jax.lax.bitcast_convert_type(s, jnp.int32)
    i = jnp.int32(0x5F3759DF) - jax.lax.shift_right_logical(i, 1)
    r = jax.lax.bitcast_convert_type(i, jnp.float32)
    for _ in range(3):
        r = r * (1.5 - 0.5 * s * r * r)
    return r


def _tanh16(x):
    x = jnp.minimum(jnp.maximum(x, -7.90531110763549805), 7.90531110763549805)
    t = x * x
    p = jnp.float32(-2.76076847742355e-16)
    for c in (2.00018790482477e-13, -8.60467152213735e-11,
              5.12229709037114e-08, 1.48572235717979e-05,
              6.37261928875436e-04, 4.89352455891786e-03):
        p = p * t + c
    p = p * x
    q = jnp.float32(1.19825839466702e-06)
    for c in (1.18534705686654e-04, 2.26843463243900e-03,
              4.89352518554385e-03):
        q = q * t + c
    return jnp.where(jnp.abs(x) < 0.0004, x, p / q)


def _bcast_lane(v, k):
    return jax.lax.broadcast_in_dim(
        jax.lax.squeeze(jax.lax.slice(v, (k,), (k + 1,)), (0,)), (16,), ())


def _round_bf16(v):
    i = jax.lax.bitcast_convert_type(v, jnp.int32)
    lsb = jax.lax.shift_right_logical(i, 16) & jnp.int32(1)
    i = i + jnp.int32(0x7FFF) + lsb
    i = i & jnp.int32(-65536)
    return jax.lax.bitcast_convert_type(i, jnp.float32)


def _sc_combine_deg(parts, z1):
    @functools.partial(
        pl.kernel,
        out_type=[jax.ShapeDtypeStruct((NPAD, W), jnp.float32),
                  jax.ShapeDtypeStruct((NPAD, W), jnp.float32)],
        mesh=_MESH,
        compiler_params=_CP,
        scratch_types=[
            pltpu.VMEM((ROWS_PER_WKR, W), jnp.float32),
            pltpu.VMEM((ROWS_PER_WKR, W), jnp.float32),
            pltpu.VMEM((ROWS_PER_WKR, W), jnp.float32),
            pltpu.VMEM((ROWS_PER_WKR, W), jnp.float32),
            pltpu.SemaphoreType.DMA,
        ],
    )
    def cdeg(parts_hbm, z_hbm, dinv_hbm, u_hbm, p0b, p1b, zb, ob, sem):
        cid, sid, wid = _worker_ids()
        stripe = pl.ds(wid * ROWS_PER_WKR, ROWS_PER_WKR)
        h0 = pltpu.async_copy(parts_hbm.at[0, stripe], p0b, sem)
        h1 = pltpu.async_copy(parts_hbm.at[1, stripe], p1b, sem)
        h2 = pltpu.async_copy(z_hbm.at[stripe], zb, sem)
        h0.wait(); h1.wait(); h2.wait()

        @pl.loop(0, ROWS_PER_WKR)
        def _(i):
            s = p0b[i, :] + p1b[i, :] + 1.0
            r = _nrsqrt(s)
            p0b[i, :] = r
            ob[i, :] = r * zb[i, :]

        pltpu.sync_copy(p0b, dinv_hbm.at[stripe])
        pltpu.sync_copy(ob, u_hbm.at[stripe])

    return cdeg(parts, z1)


def _sc_combine_layer(parts, uprev, dinv, Wp, bp, nk, final=False, bc2=None):
    n_out = 2 if final else 1
    out_t = [jax.ShapeDtypeStruct((NPAD, W), jnp.float32)] * n_out

    @functools.partial(
        pl.kernel,
        out_type=out_t if final else out_t[0],
        mesh=_MESH,
        compiler_params=_CP,
        scratch_types=[
            pltpu.VMEM((ROWS_PER_WKR, W), jnp.float32),
            pltpu.VMEM((ROWS_PER_WKR, W), jnp.float32),
            pltpu.VMEM((ROWS_PER_WKR, W), jnp.float32),
            pltpu.VMEM((ROWS_PER_WKR, W), jnp.float32),
            pltpu.VMEM((ROWS_PER_WKR, W), jnp.float32),
            pltpu.VMEM((16, W), jnp.float32),
            pltpu.VMEM((2, W), jnp.float32),
            pltpu.SemaphoreType.DMA,
        ],
    )
    def clayer(parts_hbm, u_hbm, dinv_hbm, w_hbm, b_hbm, *rest):
        if final:
            o1_hbm, o2_hbm, p0b, p1b, ub, db, ob, wb, bb, sem = rest
        else:
            o1_hbm, p0b, p1b, ub, db, ob, wb, bb, sem = rest
            o2_hbm = None
        cid, sid, wid = _worker_ids()
        stripe = pl.ds(wid * ROWS_PER_WKR, ROWS_PER_WKR)
        h0 = pltpu.async_copy(parts_hbm.at[0, stripe], p0b, sem)
        h1 = pltpu.async_copy(parts_hbm.at[1, stripe], p1b, sem)
        h2 = pltpu.async_copy(u_hbm.at[stripe], ub, sem)
        h3 = pltpu.async_copy(dinv_hbm.at[stripe], db, sem)
        h4 = pltpu.async_copy(w_hbm, wb, sem)
        h5 = pltpu.async_copy(b_hbm, bb, sem)
        h0.wait(); h1.wait(); h2.wait(); h3.wait(); h4.wait(); h5.wait()

        wr = [_round_bf16(wb[k, :]) for k in range(nk)]
        brow = bb[0, :]
        bcrow = bb[1, :]

        @pl.loop(0, ROWS_PER_WKR)
        def _(i):
            d = db[i, :]
            y = d * (p0b[i, :] + p1b[i, :] + ub[i, :]) + brow
            h = _tanh16(y)
            hb = _round_bf16(h)
            z = _bcast_lane(hb, 0) * wr[0]
            for k in range(1, nk):
                z = z + _bcast_lane(hb, k) * wr[k]
            if final:
                ob[i, :] = z + bcrow
                ub[i, :] = h
            else:
                ob[i, :] = d * z

        pltpu.sync_copy(ob, o1_hbm.at[stripe])
        if final:
            pltpu.sync_copy(ub, o2_hbm.at[stripe])

    b2 = bc2 if bc2 is not None else jnp.zeros((W,), jnp.float32)
    bstack = jnp.stack([bp, b2])
    return clayer(parts, uprev, dinv, Wp, bstack)


def _tc_call(body, out_shapes, *args):
    return pl.pallas_call(body, out_shape=out_shapes)(*args)


def _mm_body(x_ref, w_ref, o_ref):
    o_ref[0:N, :] = jnp.dot(x_ref[...], w_ref[...],
                            preferred_element_type=jnp.float32)
    o_ref[N:NPAD, :] = jnp.zeros((NPAD - N, W), jnp.float32)


def _deg_body(dp_ref, z_ref, dinv_ref, u_ref):
    deg = dp_ref[0] + dp_ref[1] + 1.0
    dinv = lax.rsqrt(deg)
    dinv_ref[...] = dinv
    u_ref[...] = dinv * z_ref[...]


def _layer_body(p_ref, u_ref, dinv_ref, w_ref, b_ref, un_ref):
    h = jnp.tanh(dinv_ref[...] * (p_ref[0] + p_ref[1] + u_ref[...])
                 + b_ref[...])
    un_ref[...] = dinv_ref[...] * jnp.dot(h, w_ref[...],
                                          preferred_element_type=jnp.float32)


def _final_body(p_ref, u_ref, dinv_ref, b_ref, wc_ref, bc_ref, out_ref, h_ref):
    h = jnp.tanh(dinv_ref[...] * (p_ref[0] + p_ref[1] + u_ref[...])
                 + b_ref[...])
    h_ref[...] = h[0:N, 0:2]
    out_ref[...] = jnp.dot(h[0:N], wc_ref[...],
                           preferred_element_type=jnp.float32) + bc_ref[...]


def _padw(w):
    return jnp.pad(w, ((0, 16 - w.shape[0]), (0, 16 - w.shape[1])))


def kernel(x, edge_index, W1, b1, W2, b2, W3, b3, Wc, bc):
    f32 = jnp.float32
    ei3 = edge_index.reshape(2, EROWS, 128)

    W1p = jnp.pad(W1, ((0, 0), (0, W - W1.shape[1])))
    W2p = _padw(W2)
    W3p = _padw(W3)
    Wcp = _padw(Wc)
    b1p = jnp.pad(b1, (0, W - b1.shape[0]))
    b2p = jnp.pad(b2, (0, W - b2.shape[0]))
    b3p = jnp.pad(b3, (0, W - b3.shape[0]))
    bcp = jnp.pad(bc, (0, W - bc.shape[0]))

    sds = jax.ShapeDtypeStruct

    z1 = _tc_call(_mm_body, sds((NPAD, W), f32), x, W1p)

    degp = _sc_deg(ei3)
    dinv, u1 = _sc_combine_deg(degp, z1)

    p1 = _sc_agg(u1, ei3)
    u2 = _sc_combine_layer(p1, u1, dinv, W2p, b1p, 4)

    p2 = _sc_agg(u2, ei3)
    u3 = _sc_combine_layer(p2, u2, dinv, W3p, b2p, 4)

    p3 = _sc_agg(u3, ei3)
    out16, h16 = _sc_combine_layer(p3, u3, dinv, Wcp, b3p, 2,
                                   final=True, bc2=bcp)

    return (out16[:N, :NUM_OUT], h16[:N, :2])
